# baseline (speedup 1.0000x reference)
import functools

import jax
import jax.numpy as jnp
from jax import lax
from jax.experimental import pallas as pl
from jax.experimental.pallas import tpu as pltpu
from jax.experimental.pallas import tpu_sc as plsc

N = 50000
NP = 50176
HALF = 25088
QTR = 12544
H = 64
TW = 128
EG_PAD = 307200
EPT_G = 19200
BK = 32
E_PAD = 802816
EPT_E = 25088
ACC_ROWS = 12672
BE = 1024

_f32 = jnp.float32
_i32 = jnp.int32


def _mesh():
    return plsc.VectorSubcoreMesh(core_axis_name="c", subcore_axis_name="s")


def _gat_body(T_h, ad_h, src_h, dst_h, znum_h, acc_h,
              acc, ad_v, src_v, dst_v, sidx_v, gath_v, gsem):
    c = lax.axis_index("c")
    s = lax.axis_index("s")
    ebase = s * EPT_G

    for q in range(2):
        r0 = c * HALF + q * QTR

        pltpu.sync_copy(znum_h.at[pl.ds(s * 792, 792), :],
                        acc.at[pl.ds(s * 792, 792), :])
        pltpu.sync_copy(ad_h.at[pl.ds(r0, QTR)], ad_v)
        plsc.subcore_barrier()

        pltpu.sync_copy(src_h.at[pl.ds(ebase, BK)], src_v.at[0])
        pltpu.sync_copy(dst_h.at[pl.ds(ebase, BK)], dst_v.at[0])
        pltpu.async_copy(T_h.at[src_v.at[0]], gath_v.at[0], gsem)

        def outer(m, carry):
            for i in range(2):
                g = m * 2 + i
                pltpu.make_async_copy(T_h.at[pl.ds(0, BK), :], gath_v.at[i],
                                      gsem).wait()

                @pl.when(g < 599)
                def _():
                    nb = ebase + (g + 1) * BK
                    pltpu.sync_copy(src_h.at[pl.ds(nb, BK)], src_v.at[1 - i])
                    pltpu.sync_copy(dst_h.at[pl.ds(nb, BK)], dst_v.at[1 - i])
                    pltpu.async_copy(T_h.at[src_v.at[1 - i]],
                                     gath_v.at[1 - i], gsem)

                ivec = jnp.full((16,), i, _i32)
                for j in range(BK // 16):
                    rows16 = lax.iota(_i32, 16) + j * 16
                    dv = dst_v[i, pl.ds(j * 16, 16)]
                    lid = dv - r0
                    inr = jnp.logical_and(lid >= 0, lid < QTR)
                    lidc = jnp.clip(lid, 0, QTR - 1)
                    adv = plsc.load_gather(ad_v, [lidc])
                    asv = plsc.load_gather(
                        gath_v, [ivec, rows16, jnp.full((16,), H, _i32)])
                    xv = asv + adv
                    w = jnp.exp(jnp.maximum(xv, 0.2 * xv))
                    isc = jnp.where(inr, lid, QTR)
                    sidx_v[i, pl.ds(j * 16, 16)] = isc
                    for col in range(H):
                        cvec = jnp.full((16,), col, _i32)
                        vals = plsc.load_gather(
                            gath_v, [ivec, rows16, cvec]) * w
                        plsc.store_scatter(gath_v, [ivec, rows16, cvec],
                                           vals)
                    plsc.store_scatter(
                        gath_v, [ivec, rows16, jnp.full((16,), H, _i32)], w)

                pltpu.sync_copy(gath_v.at[i], acc.at[sidx_v.at[i]], add=True)
            return carry

        lax.fori_loop(0, 300, outer, 0)

        plsc.subcore_barrier()

        pltpu.sync_copy(acc.at[pl.ds(s * 784, 784), :],
                        acc_h.at[pl.ds(r0 + s * 784, 784), :])
        plsc.subcore_barrier()


def _gat_sc(T, ad, src, dst, znum):
    f = pl.kernel(
        _gat_body,
        out_type=jax.ShapeDtypeStruct((NP, TW), _f32),
        mesh=_mesh(),
        compiler_params=pltpu.CompilerParams(needs_layout_passes=False),
        scratch_types=[
            pltpu.VMEM_SHARED((ACC_ROWS, TW), _f32),
            pltpu.VMEM((QTR,), _f32),
            pltpu.VMEM((2, BK), _i32),
            pltpu.VMEM((2, BK), _i32),
            pltpu.VMEM((2, BK), _i32),
            pltpu.VMEM((2, BK, TW), _f32),
            pltpu.SemaphoreType.DMA,
        ],
    )
    return f(T, ad, src, dst, znum)


def _pair_body(H3_h, src_h, dst_h, esrc_h, edst_h, idx_v, bufs, gsem, wsem):
    c = lax.axis_index("c")
    s = lax.axis_index("s")
    wid = s * 2 + c
    base = wid * EPT_E

    for idx_h, out_h in ((src_h, esrc_h), (dst_h, edst_h)):
        pltpu.sync_copy(idx_h.at[pl.ds(base, EPT_E)], idx_v)

        def grp(gidx, p):
            @pl.when(gidx >= 2)
            def _():
                for b in range(2):
                    pltpu.make_async_copy(H3_h.at[pl.ds(0, 128), :],
                                          bufs.at[p, b], wsem).wait()
            descs = []
            for b in range(2):
                blk = gidx * 2 + b
                descs.append(pltpu.async_copy(
                    H3_h.at[idx_v.at[pl.ds(blk * 128, 128)]],
                    bufs.at[p, b], gsem))
            for d in descs:
                d.wait()
            for b in range(2):
                blk = gidx * 2 + b
                pltpu.async_copy(
                    bufs.at[p, b],
                    out_h.at[pl.ds(base + blk * 128, 128), :], wsem)

        def outer(m, carry):
            grp(m * 2, 0)
            grp(m * 2 + 1, 1)
            return carry

        lax.fori_loop(0, 49, outer, 0)
        for b in range(4):
            pltpu.make_async_copy(H3_h.at[pl.ds(0, 128), :],
                                  bufs.at[b // 2, b % 2], wsem).wait()


def _pair_gather_sc(H3, srcp, dstp):
    f = pl.kernel(
        _pair_body,
        out_type=(jax.ShapeDtypeStruct((E_PAD, TW), _f32),
                  jax.ShapeDtypeStruct((E_PAD, TW), _f32)),
        mesh=_mesh(),
        compiler_params=pltpu.CompilerParams(needs_layout_passes=False),
        scratch_types=[
            pltpu.VMEM((EPT_E,), _i32),
            pltpu.VMEM((2, 2, 128, TW), _f32),
            pltpu.SemaphoreType.DMA,
            pltpu.SemaphoreType.DMA,
        ],
    )
    return f(H3, srcp, dstp)


RB = 7168


def _prep1_body(x_ref, w_ref, as_ref, ad_ref, t_ref, adv_ref):
    hmat = jnp.dot(x_ref[...], w_ref[...], preferred_element_type=_f32,
                   precision=lax.Precision.HIGHEST)
    as_col = jnp.sum(hmat * as_ref[...][None, :], axis=1, keepdims=True)
    t_ref[...] = jnp.concatenate(
        [hmat, as_col, jnp.zeros((RB, TW - H - 1), _f32)], axis=1)
    adv_ref[...] = jnp.sum(hmat * ad_ref[...][None, :], axis=1,
                           keepdims=True)


def _prep1(xp, W1, a_src1, a_dst1):
    return pl.pallas_call(
        _prep1_body,
        grid=(NP // RB,),
        in_specs=[
            pl.BlockSpec((RB, 3), lambda i: (i, 0)),
            pl.BlockSpec((3, H), lambda i: (0, 0)),
            pl.BlockSpec((H,), lambda i: (0,)),
            pl.BlockSpec((H,), lambda i: (0,)),
        ],
        out_specs=[
            pl.BlockSpec((RB, TW), lambda i: (i, 0)),
            pl.BlockSpec((RB, 1), lambda i: (i, 0)),
        ],
        out_shape=[
            jax.ShapeDtypeStruct((NP, TW), _f32),
            jax.ShapeDtypeStruct((NP, 1), _f32),
        ],
    )(xp, W1, a_src1, a_dst1)


def _elu(x):
    return jnp.where(x > 0, x, jnp.exp(jnp.minimum(x, 0.0)) - 1.0)


def _prep2_body(acc_ref, b_ref, w_ref, as_ref, ad_ref, t_ref, adv_ref):
    accv = acc_ref[...]
    hprev = _elu(accv[:, :H] / (accv[:, H:H + 1] + 1e-16)
                 + b_ref[...][None, :])
    hmat = jnp.dot(hprev, w_ref[...], preferred_element_type=_f32,
                   precision=lax.Precision.HIGHEST)
    as_col = jnp.sum(hmat * as_ref[...][None, :], axis=1, keepdims=True)
    t_ref[...] = jnp.concatenate(
        [hmat, as_col, jnp.zeros((RB, TW - H - 1), _f32)], axis=1)
    adv_ref[...] = jnp.sum(hmat * ad_ref[...][None, :], axis=1,
                           keepdims=True)


def _prep2(acc1, b1, W2, a_src2, a_dst2):
    return pl.pallas_call(
        _prep2_body,
        grid=(NP // RB,),
        in_specs=[
            pl.BlockSpec((RB, TW), lambda i: (i, 0)),
            pl.BlockSpec((H,), lambda i: (0,)),
            pl.BlockSpec((H, H), lambda i: (0, 0)),
            pl.BlockSpec((H,), lambda i: (0,)),
            pl.BlockSpec((H,), lambda i: (0,)),
        ],
        out_specs=[
            pl.BlockSpec((RB, TW), lambda i: (i, 0)),
            pl.BlockSpec((RB, 1), lambda i: (i, 0)),
        ],
        out_shape=[
            jax.ShapeDtypeStruct((NP, TW), _f32),
            jax.ShapeDtypeStruct((NP, 1), _f32),
        ],
    )(acc1, b1, W2, a_src2, a_dst2)


def _prep3_body(acc_ref, b_ref, h_ref):
    accv = acc_ref[...]
    hv = _elu(accv[:, :H] / (accv[:, H:H + 1] + 1e-16)
              + b_ref[...][None, :])
    h_ref[...] = jnp.concatenate(
        [hv, jnp.zeros((RB, TW - H), _f32)], axis=1)


def _prep3(acc2, b2):
    return pl.pallas_call(
        _prep3_body,
        grid=(NP // RB,),
        in_specs=[
            pl.BlockSpec((RB, TW), lambda i: (i, 0)),
            pl.BlockSpec((H,), lambda i: (0,)),
        ],
        out_specs=pl.BlockSpec((RB, TW), lambda i: (i, 0)),
        out_shape=jax.ShapeDtypeStruct((NP, TW), _f32),
    )(acc2, b2)


def _bf(v):
    return v.astype(jnp.bfloat16)


def _mlp_body(esrc_ref, edst_ref, attr_ref, wa_ref, wb_ref, wc_ref, w2_ref,
              out_ref):
    attr = attr_ref[...]
    ones = jnp.ones((attr.shape[0], 1), _f32)
    attr_aug = jnp.concatenate([attr, ones], axis=1)
    hid = (jnp.dot(_bf(esrc_ref[...]), _bf(wa_ref[...]),
                   preferred_element_type=_f32)
           + jnp.dot(_bf(edst_ref[...]), _bf(wb_ref[...]),
                     preferred_element_type=_f32)
           + jnp.dot(_bf(attr_aug), _bf(wc_ref[...]),
                     preferred_element_type=_f32))
    hid = _elu(hid)
    hid_b = _bf(hid).astype(_f32)
    w2_b = _bf(w2_ref[...]).astype(_f32)
    out_ref[...] = jnp.sum(hid_b * w2_b, axis=1)


def _mlp_pallas(esrc, edst, attr, wa, wb, wc, w2full):
    return pl.pallas_call(
        _mlp_body,
        grid=(E_PAD // BE,),
        in_specs=[
            pl.BlockSpec((BE, TW), lambda i: (i, 0)),
            pl.BlockSpec((BE, TW), lambda i: (i, 0)),
            pl.BlockSpec((BE, 2), lambda i: (i, 0)),
            pl.BlockSpec((TW, 521), lambda i: (0, 0)),
            pl.BlockSpec((TW, 521), lambda i: (0, 0)),
            pl.BlockSpec((3, 521), lambda i: (0, 0)),
            pl.BlockSpec((1, 521), lambda i: (0, 0)),
        ],
        out_specs=pl.BlockSpec((BE,), lambda i: (i,)),
        out_shape=jax.ShapeDtypeStruct((E_PAD,), _f32),
    )(esrc, edst, attr, wa, wb, wc, w2full)


def kernel(x, edge_index, edge_attr, global_edge_index,
           W1, a_src1, a_dst1, b1, W2, a_src2, a_dst2, b2,
           Wm1, bm1, Wm2, bm2):
    E = edge_index.shape[1]
    EG = global_edge_index.shape[1]

    xp = jnp.concatenate([x, jnp.zeros((NP - N, x.shape[1]), _f32)], axis=0)
    srcg = jnp.concatenate(
        [global_edge_index[0], jnp.zeros(EG_PAD - EG, _i32)])
    dstg = jnp.concatenate(
        [global_edge_index[1], jnp.full(EG_PAD - EG, NP - 1, _i32)])
    srce = jnp.concatenate([edge_index[0], jnp.zeros(E_PAD - E, _i32)])
    dste = jnp.concatenate([edge_index[1], jnp.zeros(E_PAD - E, _i32)])
    attrp = jnp.concatenate(
        [edge_attr, jnp.zeros((E_PAD - E, 2), _f32)], axis=0)
    znum = jnp.zeros((ACC_ROWS, TW), _f32)

    T1, ad1 = _prep1(xp, W1, a_src1, a_dst1)
    acc1 = _gat_sc(T1, ad1.reshape(NP), srcg, dstg, znum)
    T2, ad2 = _prep2(acc1, b1, W2, a_src2, a_dst2)
    acc2 = _gat_sc(T2, ad2.reshape(NP), srcg, dstg, znum)
    H3 = _prep3(acc2, b2)
    esrc, edst = _pair_gather_sc(H3, srce, dste)

    zpad = jnp.zeros((TW - H, 520), _f32)
    zcol = jnp.zeros((TW, 1), _f32)
    wa = jnp.concatenate(
        [jnp.concatenate([Wm1[:H], zpad], axis=0), zcol], axis=1)
    wb = jnp.concatenate(
        [jnp.concatenate([Wm1[H:2 * H], zpad], axis=0), zcol], axis=1)
    wc_main = jnp.concatenate([Wm1[2 * H:], bm1[None, :]], axis=0)
    wc_col = jnp.zeros((3, 1), _f32).at[2, 0].set(1.0)
    wc = jnp.concatenate([wc_main, wc_col], axis=1)
    w2full = jnp.concatenate([Wm2[:, 0], bm2])[None, :]

    out = _mlp_pallas(esrc, edst, attrp, wa, wb, wc, w2full)
    return out[:E]


# speedup vs baseline: 1.1507x; 1.1507x over previous
import functools

import jax
import jax.numpy as jnp
from jax import lax
from jax.experimental import pallas as pl
from jax.experimental.pallas import tpu as pltpu
from jax.experimental.pallas import tpu_sc as plsc

N = 50000
NP = 50176
HALF = 25088
QTR = 12544
H = 64
TW = 128
EG_PAD = 307200
EPT_G = 19200
BK = 32
E_PAD = 802816
EPT_E = 25088
ACC_ROWS = 12672
BE = 1024

_f32 = jnp.float32
_i32 = jnp.int32


def _mesh():
    return plsc.VectorSubcoreMesh(core_axis_name="c", subcore_axis_name="s")


def _gat_body(T_h, ad_h, src_h, dst_h, znum_h, acc_h,
              acc, ad_v, src_v, dst_v, sidx_v, gath_v, gsem, ssem0, ssem1):
    c = lax.axis_index("c")
    s = lax.axis_index("s")
    ebase = s * EPT_G

    for q in range(2):
        r0 = c * HALF + q * QTR

        pltpu.sync_copy(znum_h.at[pl.ds(s * 792, 792), :],
                        acc.at[pl.ds(s * 792, 792), :])
        pltpu.sync_copy(ad_h.at[pl.ds(r0, QTR)], ad_v)
        plsc.subcore_barrier()

        pltpu.sync_copy(src_h.at[pl.ds(ebase, 8 * BK)], src_v.at[0])
        pltpu.sync_copy(dst_h.at[pl.ds(ebase, 8 * BK)], dst_v.at[0])
        pltpu.async_copy(T_h.at[src_v.at[0, pl.ds(0, BK)]], gath_v.at[0],
                         gsem)

        sems = (ssem0, ssem1)

        def outer(m, carry):
            ch = m & 1

            @pl.when(m < 74)
            def _():
                nb = ebase + (m + 1) * 8 * BK
                pltpu.sync_copy(src_h.at[pl.ds(nb, 8 * BK)],
                                src_v.at[1 - ch])
                pltpu.sync_copy(dst_h.at[pl.ds(nb, 8 * BK)],
                                dst_v.at[1 - ch])

            for b in range(8):
                i = b & 1
                g = m * 8 + b
                pltpu.make_async_copy(T_h.at[pl.ds(0, BK), :], gath_v.at[i],
                                      gsem).wait()

                @pl.when(g < 599)
                def _():
                    @pl.when(g >= 1)
                    def _():
                        pltpu.make_async_copy(
                            T_h.at[pl.ds(0, BK), :], gath_v.at[1 - i],
                            sems[1 - i]).wait()
                    bn = b + 1
                    cn = ch if bn < 8 else 1 - ch
                    bo = (bn % 8) * BK
                    pltpu.async_copy(
                        T_h.at[src_v.at[cn, pl.ds(bo, BK)]],
                        gath_v.at[1 - i], gsem)

                ivec = jnp.full((16,), i, _i32)
                chvec = jnp.full((16,), ch, _i32)
                for j in range(BK // 16):
                    rows16 = lax.iota(_i32, 16) + j * 16
                    dv = plsc.load_gather(
                        dst_v, [chvec, rows16 + b * BK])
                    lid = dv - r0
                    inr = jnp.logical_and(lid >= 0, lid < QTR)
                    lidc = jnp.clip(lid, 0, QTR - 1)
                    adv = plsc.load_gather(ad_v, [lidc])
                    asv = plsc.load_gather(
                        gath_v, [ivec, rows16, jnp.full((16,), H, _i32)])
                    xv = asv + adv
                    w = jnp.exp(jnp.maximum(xv, 0.2 * xv))
                    isc = jnp.where(inr, lid, QTR)
                    sidx_v[i, pl.ds(j * 16, 16)] = isc
                    for col in range(H):
                        cvec = jnp.full((16,), col, _i32)
                        vals = plsc.load_gather(
                            gath_v, [ivec, rows16, cvec]) * w
                        plsc.store_scatter(gath_v, [ivec, rows16, cvec],
                                           vals)
                    plsc.store_scatter(
                        gath_v, [ivec, rows16, jnp.full((16,), H, _i32)], w)

                pltpu.async_copy(gath_v.at[i], acc.at[sidx_v.at[i]],
                                 sems[i], add=True)
            return carry

        lax.fori_loop(0, 75, outer, 0)

        for i in range(2):
            pltpu.make_async_copy(T_h.at[pl.ds(0, BK), :], gath_v.at[i],
                                  sems[i]).wait()
        plsc.subcore_barrier()

        pltpu.sync_copy(acc.at[pl.ds(s * 784, 784), :],
                        acc_h.at[pl.ds(r0 + s * 784, 784), :])
        plsc.subcore_barrier()


def _gat_sc(T, ad, src, dst, znum):
    f = pl.kernel(
        _gat_body,
        out_type=jax.ShapeDtypeStruct((NP, TW), _f32),
        mesh=_mesh(),
        compiler_params=pltpu.CompilerParams(needs_layout_passes=False),
        scratch_types=[
            pltpu.VMEM_SHARED((ACC_ROWS, TW), _f32),
            pltpu.VMEM((QTR,), _f32),
            pltpu.VMEM((2, 8 * BK), _i32),
            pltpu.VMEM((2, 8 * BK), _i32),
            pltpu.VMEM((2, BK), _i32),
            pltpu.VMEM((2, BK, TW), _f32),
            pltpu.SemaphoreType.DMA,
            pltpu.SemaphoreType.DMA,
            pltpu.SemaphoreType.DMA,
        ],
    )
    return f(T, ad, src, dst, znum)


def _pair_body(H3_h, src_h, dst_h, esrc_h, edst_h, idx_v, bufs, gsem, wsem):
    c = lax.axis_index("c")
    s = lax.axis_index("s")
    wid = s * 2 + c
    base = wid * EPT_E

    for idx_h, out_h in ((src_h, esrc_h), (dst_h, edst_h)):
        pltpu.sync_copy(idx_h.at[pl.ds(base, EPT_E)], idx_v)

        def grp(gidx, p):
            @pl.when(gidx >= 2)
            def _():
                for b in range(2):
                    pltpu.make_async_copy(H3_h.at[pl.ds(0, 128), :],
                                          bufs.at[p, b], wsem).wait()
            descs = []
            for b in range(2):
                blk = gidx * 2 + b
                descs.append(pltpu.async_copy(
                    H3_h.at[idx_v.at[pl.ds(blk * 128, 128)]],
                    bufs.at[p, b], gsem))
            for d in descs:
                d.wait()
            for b in range(2):
                blk = gidx * 2 + b
                pltpu.async_copy(
                    bufs.at[p, b],
                    out_h.at[pl.ds(base + blk * 128, 128), :], wsem)

        def outer(m, carry):
            grp(m * 2, 0)
            grp(m * 2 + 1, 1)
            return carry

        lax.fori_loop(0, 49, outer, 0)
        for b in range(4):
            pltpu.make_async_copy(H3_h.at[pl.ds(0, 128), :],
                                  bufs.at[b // 2, b % 2], wsem).wait()


def _pair_gather_sc(H3, srcp, dstp):
    f = pl.kernel(
        _pair_body,
        out_type=(jax.ShapeDtypeStruct((E_PAD, TW), _f32),
                  jax.ShapeDtypeStruct((E_PAD, TW), _f32)),
        mesh=_mesh(),
        compiler_params=pltpu.CompilerParams(needs_layout_passes=False),
        scratch_types=[
            pltpu.VMEM((EPT_E,), _i32),
            pltpu.VMEM((2, 2, 128, TW), _f32),
            pltpu.SemaphoreType.DMA,
            pltpu.SemaphoreType.DMA,
        ],
    )
    return f(H3, srcp, dstp)


RB = 7168


def _prep1_body(x_ref, w_ref, as_ref, ad_ref, t_ref, adv_ref):
    hmat = jnp.dot(x_ref[...], w_ref[...], preferred_element_type=_f32,
                   precision=lax.Precision.HIGHEST)
    as_col = jnp.sum(hmat * as_ref[...][None, :], axis=1, keepdims=True)
    t_ref[...] = jnp.concatenate(
        [hmat, as_col, jnp.zeros((RB, TW - H - 1), _f32)], axis=1)
    adv_ref[...] = jnp.sum(hmat * ad_ref[...][None, :], axis=1,
                           keepdims=True)


def _prep1(xp, W1, a_src1, a_dst1):
    return pl.pallas_call(
        _prep1_body,
        grid=(NP // RB,),
        in_specs=[
            pl.BlockSpec((RB, 3), lambda i: (i, 0)),
            pl.BlockSpec((3, H), lambda i: (0, 0)),
            pl.BlockSpec((H,), lambda i: (0,)),
            pl.BlockSpec((H,), lambda i: (0,)),
        ],
        out_specs=[
            pl.BlockSpec((RB, TW), lambda i: (i, 0)),
            pl.BlockSpec((RB, 1), lambda i: (i, 0)),
        ],
        out_shape=[
            jax.ShapeDtypeStruct((NP, TW), _f32),
            jax.ShapeDtypeStruct((NP, 1), _f32),
        ],
    )(xp, W1, a_src1, a_dst1)


def _elu(x):
    return jnp.where(x > 0, x, jnp.exp(jnp.minimum(x, 0.0)) - 1.0)


def _prep2_body(acc_ref, b_ref, w_ref, as_ref, ad_ref, t_ref, adv_ref):
    accv = acc_ref[...]
    hprev = _elu(accv[:, :H] / (accv[:, H:H + 1] + 1e-16)
                 + b_ref[...][None, :])
    hmat = jnp.dot(hprev, w_ref[...], preferred_element_type=_f32,
                   precision=lax.Precision.HIGHEST)
    as_col = jnp.sum(hmat * as_ref[...][None, :], axis=1, keepdims=True)
    t_ref[...] = jnp.concatenate(
        [hmat, as_col, jnp.zeros((RB, TW - H - 1), _f32)], axis=1)
    adv_ref[...] = jnp.sum(hmat * ad_ref[...][None, :], axis=1,
                           keepdims=True)


def _prep2(acc1, b1, W2, a_src2, a_dst2):
    return pl.pallas_call(
        _prep2_body,
        grid=(NP // RB,),
        in_specs=[
            pl.BlockSpec((RB, TW), lambda i: (i, 0)),
            pl.BlockSpec((H,), lambda i: (0,)),
            pl.BlockSpec((H, H), lambda i: (0, 0)),
            pl.BlockSpec((H,), lambda i: (0,)),
            pl.BlockSpec((H,), lambda i: (0,)),
        ],
        out_specs=[
            pl.BlockSpec((RB, TW), lambda i: (i, 0)),
            pl.BlockSpec((RB, 1), lambda i: (i, 0)),
        ],
        out_shape=[
            jax.ShapeDtypeStruct((NP, TW), _f32),
            jax.ShapeDtypeStruct((NP, 1), _f32),
        ],
    )(acc1, b1, W2, a_src2, a_dst2)


def _prep3_body(acc_ref, b_ref, h_ref):
    accv = acc_ref[...]
    hv = _elu(accv[:, :H] / (accv[:, H:H + 1] + 1e-16)
              + b_ref[...][None, :])
    h_ref[...] = jnp.concatenate(
        [hv, jnp.zeros((RB, TW - H), _f32)], axis=1)


def _prep3(acc2, b2):
    return pl.pallas_call(
        _prep3_body,
        grid=(NP // RB,),
        in_specs=[
            pl.BlockSpec((RB, TW), lambda i: (i, 0)),
            pl.BlockSpec((H,), lambda i: (0,)),
        ],
        out_specs=pl.BlockSpec((RB, TW), lambda i: (i, 0)),
        out_shape=jax.ShapeDtypeStruct((NP, TW), _f32),
    )(acc2, b2)


def _bf(v):
    return v.astype(jnp.bfloat16)


def _mlp_body(esrc_ref, edst_ref, attr_ref, wa_ref, wb_ref, wc_ref, w2_ref,
              out_ref):
    attr = attr_ref[...]
    ones = jnp.ones((attr.shape[0], 1), _f32)
    attr_aug = jnp.concatenate([attr, ones], axis=1)
    hid = (jnp.dot(_bf(esrc_ref[...]), _bf(wa_ref[...]),
                   preferred_element_type=_f32)
           + jnp.dot(_bf(edst_ref[...]), _bf(wb_ref[...]),
                     preferred_element_type=_f32)
           + jnp.dot(_bf(attr_aug), _bf(wc_ref[...]),
                     preferred_element_type=_f32))
    hid = _elu(hid)
    hid_b = _bf(hid).astype(_f32)
    w2_b = _bf(w2_ref[...]).astype(_f32)
    out_ref[...] = jnp.sum(hid_b * w2_b, axis=1)


def _mlp_pallas(esrc, edst, attr, wa, wb, wc, w2full):
    return pl.pallas_call(
        _mlp_body,
        grid=(E_PAD // BE,),
        in_specs=[
            pl.BlockSpec((BE, TW), lambda i: (i, 0)),
            pl.BlockSpec((BE, TW), lambda i: (i, 0)),
            pl.BlockSpec((BE, 2), lambda i: (i, 0)),
            pl.BlockSpec((TW, 521), lambda i: (0, 0)),
            pl.BlockSpec((TW, 521), lambda i: (0, 0)),
            pl.BlockSpec((3, 521), lambda i: (0, 0)),
            pl.BlockSpec((1, 521), lambda i: (0, 0)),
        ],
        out_specs=pl.BlockSpec((BE,), lambda i: (i,)),
        out_shape=jax.ShapeDtypeStruct((E_PAD,), _f32),
    )(esrc, edst, attr, wa, wb, wc, w2full)


def kernel(x, edge_index, edge_attr, global_edge_index,
           W1, a_src1, a_dst1, b1, W2, a_src2, a_dst2, b2,
           Wm1, bm1, Wm2, bm2):
    E = edge_index.shape[1]
    EG = global_edge_index.shape[1]

    xp = jnp.concatenate([x, jnp.zeros((NP - N, x.shape[1]), _f32)], axis=0)
    srcg = jnp.concatenate(
        [global_edge_index[0], jnp.zeros(EG_PAD - EG, _i32)])
    dstg = jnp.concatenate(
        [global_edge_index[1], jnp.full(EG_PAD - EG, NP - 1, _i32)])
    srce = jnp.concatenate([edge_index[0], jnp.zeros(E_PAD - E, _i32)])
    dste = jnp.concatenate([edge_index[1], jnp.zeros(E_PAD - E, _i32)])
    attrp = jnp.concatenate(
        [edge_attr, jnp.zeros((E_PAD - E, 2), _f32)], axis=0)
    znum = jnp.zeros((ACC_ROWS, TW), _f32)

    T1, ad1 = _prep1(xp, W1, a_src1, a_dst1)
    acc1 = _gat_sc(T1, ad1.reshape(NP), srcg, dstg, znum)
    T2, ad2 = _prep2(acc1, b1, W2, a_src2, a_dst2)
    acc2 = _gat_sc(T2, ad2.reshape(NP), srcg, dstg, znum)
    H3 = _prep3(acc2, b2)
    esrc, edst = _pair_gather_sc(H3, srce, dste)

    zpad = jnp.zeros((TW - H, 520), _f32)
    zcol = jnp.zeros((TW, 1), _f32)
    wa = jnp.concatenate(
        [jnp.concatenate([Wm1[:H], zpad], axis=0), zcol], axis=1)
    wb = jnp.concatenate(
        [jnp.concatenate([Wm1[H:2 * H], zpad], axis=0), zcol], axis=1)
    wc_main = jnp.concatenate([Wm1[2 * H:], bm1[None, :]], axis=0)
    wc_col = jnp.zeros((3, 1), _f32).at[2, 0].set(1.0)
    wc = jnp.concatenate([wc_main, wc_col], axis=1)
    w2full = jnp.concatenate([Wm2[:, 0], bm2])[None, :]

    out = _mlp_pallas(esrc, edst, attrp, wa, wb, wc, w2full)
    return out[:E]
